# Initial kernel scaffold; baseline (speedup 1.0000x reference)
import functools

import jax
import jax.numpy as jnp
from jax import lax
from jax.experimental import pallas as pl
from jax.experimental.pallas import tpu as pltpu
from jax.experimental.pallas import tpu_sc as plsc

N = 10000
E = 320000
L = 4
FEATURE_LEN = 128
DIM = 128
HIDDEN = 256

NC = 2
NS = 16
NW = NC * NS
EPW = E // NW
CHUNK = 128
NCHUNK = -(-EPW // CHUNK)
EPW_PAD = NCHUNK * CHUNK
DUMMY = N
NACC = N + 16
ROWS_PER_TILE = N // NS
ZCH = ROWS_PER_TILE - 4 * CHUNK

_sc_mesh = plsc.VectorSubcoreMesh(core_axis_name="c", subcore_axis_name="s")


@functools.partial(
    pl.kernel,
    out_type=jax.ShapeDtypeStruct((4, NW, N), jnp.float32),
    mesh=_sc_mesh,
    scratch_types=[
        pltpu.VMEM((EPW,), jnp.int32),
        pltpu.VMEM((N,), jnp.float32),
    ],
)
def _degree_kernel(s1_hbm, d1_hbm, s2_hbm, d2_hbm, out_hbm, idx_v, hist_v):
    w = lax.axis_index("c") * NS + lax.axis_index("s")
    ones = jnp.ones((16,), jnp.float32)
    zeros = jnp.zeros((16,), jnp.float32)
    for t, e_hbm in enumerate((s1_hbm, d1_hbm, s2_hbm, d2_hbm)):
        pltpu.sync_copy(e_hbm.at[w], idx_v)

        def zbody(i, _):
            hist_v[pl.ds(i * 16, 16)] = zeros
            return 0

        lax.fori_loop(0, N // 16, zbody, 0)

        def hbody(i, _):
            idx = idx_v[pl.ds(i * 16, 16)]
            plsc.addupdate_scatter(hist_v, [idx], ones)
            return 0

        lax.fori_loop(0, EPW // 16, hbody, 0)
        pltpu.sync_copy(hist_v, out_hbm.at[t, w])


@functools.partial(
    pl.kernel,
    out_type=jax.ShapeDtypeStruct((NC, N, DIM), jnp.float32),
    mesh=_sc_mesh,
    scratch_types=[
        pltpu.VMEM((NCHUNK, CHUNK), jnp.int32),
        pltpu.VMEM((NCHUNK, CHUNK), jnp.int32),
        pltpu.VMEM((CHUNK, DIM), jnp.float32),
        pltpu.VMEM_SHARED((NACC, DIM), jnp.float32),
        pltpu.SemaphoreType.DMA,
    ],
)
def _mp_kernel(h_hbm, srcp_hbm, dstp_hbm, out_hbm, src_v, dst_v, buf_v, acc_sh, sem):
    c = lax.axis_index("c")
    s = lax.axis_index("s")
    w = c * NS + s
    pltpu.sync_copy(srcp_hbm.at[w], src_v)
    pltpu.sync_copy(dstp_hbm.at[w], dst_v)

    zeros = jnp.zeros((16,), jnp.float32)

    def zbody(i, _):
        buf_v[i // 8, pl.ds((i % 8) * 16, 16)] = zeros
        return 0

    lax.fori_loop(0, CHUNK * (DIM // 16), zbody, 0)

    row0 = s * ROWS_PER_TILE
    for z in range(4):
        pltpu.sync_copy(buf_v, acc_sh.at[pl.ds(row0 + z * CHUNK, CHUNK)])
    pltpu.sync_copy(buf_v.at[pl.ds(0, ZCH)],
                    acc_sh.at[pl.ds(row0 + 4 * CHUNK, ZCH)])
    plsc.subcore_barrier()

    def mbody(j, _):
        pltpu.async_copy(h_hbm.at[src_v.at[j]], buf_v, sem).wait()
        pltpu.sync_copy(buf_v, acc_sh.at[dst_v.at[j]], add=True)
        return 0

    lax.fori_loop(0, NCHUNK, mbody, 0)
    plsc.subcore_barrier()

    for z in range(4):
        pltpu.sync_copy(acc_sh.at[pl.ds(row0 + z * CHUNK, CHUNK)],
                        out_hbm.at[c, pl.ds(row0 + z * CHUNK, CHUNK)])
    pltpu.sync_copy(acc_sh.at[pl.ds(row0 + 4 * CHUNK, ZCH)],
                    out_hbm.at[c, pl.ds(row0 + 4 * CHUNK, ZCH)])


def _tc0_body(feat_ref, degs_ref, degd_ref, w0_ref, h_out, ns_out, nd_out):
    degs = jnp.sum(degs_ref[...], axis=0)
    degd = jnp.sum(degd_ref[...], axis=0)
    ns = lax.rsqrt(jnp.maximum(degs, 1.0))
    nd = lax.rsqrt(jnp.maximum(degd, 1.0))
    f = feat_ref[...]
    iot = lax.broadcasted_iota(jnp.int32, (N, FEATURE_LEN), 1)
    h = jnp.zeros((N, FEATURE_LEN), jnp.float32)
    for j in range(L):
        h = h + (f[:, j][:, None] == iot).astype(jnp.float32)
    hs = jnp.dot(h, w0_ref[...], preferred_element_type=jnp.float32)
    h_out[...] = hs * ns[:, None]
    ns_out[...] = ns[:, None]
    nd_out[...] = nd[:, None]


_tc0 = pl.pallas_call(
    _tc0_body,
    out_shape=[
        jax.ShapeDtypeStruct((N, DIM), jnp.float32),
        jax.ShapeDtypeStruct((N, 1), jnp.float32),
        jax.ShapeDtypeStruct((N, 1), jnp.float32),
    ],
)


def _tc1_body(acc_ref, nd_ref, ns_ref, b_ref, w_ref, out_ref):
    agg = acc_ref[0] + acc_ref[1]
    h = jnp.maximum(agg * nd_ref[...] + b_ref[...][None, :], 0.0)
    out_ref[...] = jnp.dot(h, w_ref[...],
                           preferred_element_type=jnp.float32) * ns_ref[...]


_tc1 = pl.pallas_call(
    _tc1_body,
    out_shape=jax.ShapeDtypeStruct((N, DIM), jnp.float32),
)


def _tc2_body(acc_ref, nd_ref, b_ref, rs_out, nn_out):
    h = (acc_ref[0] + acc_ref[1]) * nd_ref[...] + b_ref[...][None, :]
    rs_out[...] = jnp.sum(h, axis=0, keepdims=True)
    nn = jnp.sqrt(jnp.sum(h * h, axis=1))
    nn_out[...] = jnp.sum(nn)[None, None]


_tc2 = pl.pallas_call(
    _tc2_body,
    out_shape=[
        jax.ShapeDtypeStruct((1, DIM), jnp.float32),
        jax.ShapeDtypeStruct((1, 1), jnp.float32),
    ],
)


def _tc3_body(rs1_ref, nn1_ref, rs2_ref, wh_ref, bh_ref, wo_ref, bo_ref, out_ref):
    factor = jnp.sqrt(jnp.float32(DIM)) * N / nn1_ref[0, 0]
    emb = jnp.concatenate([rs1_ref[...], rs2_ref[...]], axis=1) * factor
    hid = jnp.maximum(
        jnp.dot(emb, wh_ref[...], preferred_element_type=jnp.float32)
        + bh_ref[...][None, :], 0.0)
    out_ref[...] = (jnp.dot(hid, wo_ref[...], preferred_element_type=jnp.float32)
                    + bo_ref[...][None, :])


_tc3 = pl.pallas_call(
    _tc3_body,
    out_shape=jax.ShapeDtypeStruct((1, 1), jnp.float32),
)


def _prep_edges(edge_index):
    src = edge_index[0].astype(jnp.int32).reshape(NW, EPW)
    dst = edge_index[1].astype(jnp.int32).reshape(NW, EPW)
    pad = EPW_PAD - EPW
    srcp = jnp.pad(src, ((0, 0), (0, pad)),
                   constant_values=0).reshape(NW, NCHUNK, CHUNK)
    dstp = jnp.pad(dst, ((0, 0), (0, pad)),
                   constant_values=DUMMY).reshape(NW, NCHUNK, CHUNK)
    return src, dst, srcp, dstp


def kernel(feature1, edge_index1, feature2, edge_index2,
           W0, b0, W1, b1, Wh, bh, Wo, bo):
    src1, dst1, srcp1, dstp1 = _prep_edges(edge_index1)
    src2, dst2, srcp2, dstp2 = _prep_edges(edge_index2)

    degp = _degree_kernel(src1, dst1, src2, dst2)

    hs0_1, ns1, nd1 = _tc0(feature1.astype(jnp.int32), degp[0], degp[1], W0)
    hs0_2, ns2, nd2 = _tc0(feature2.astype(jnp.int32), degp[2], degp[3], W0)

    acc1 = _mp_kernel(hs0_1, srcp1, dstp1)
    hs1_1 = _tc1(acc1, nd1, ns1, b0, W1)
    acc1b = _mp_kernel(hs1_1, srcp1, dstp1)
    rs1, nn1 = _tc2(acc1b, nd1, b1)

    acc2 = _mp_kernel(hs0_2, srcp2, dstp2)
    hs1_2 = _tc1(acc2, nd2, ns2, b0, W1)
    acc2b = _mp_kernel(hs1_2, srcp2, dstp2)
    rs2, nn2 = _tc2(acc2b, nd2, b1)

    return _tc3(rs1, nn1, rs2, Wh, bh, Wo, bo)


# speedup vs baseline: 4.0246x; 4.0246x over previous
import functools

import jax
import jax.numpy as jnp
from jax import lax
from jax.experimental import pallas as pl
from jax.experimental.pallas import tpu as pltpu
from jax.experimental.pallas import tpu_sc as plsc

N = 10000
E = 320000
L = 4
FEATURE_LEN = 128
DIM = 128
HIDDEN = 256

NC = 2
NS = 16
NW = NC * NS
EPW = E // NW
CHUNK = 128
NCHUNK = -(-EPW // CHUNK)
EPW_PAD = NCHUNK * CHUNK
DUMMY = N
NPAD = 10240
ROWS_PER_TILE = NPAD // NS
RCH = ROWS_PER_TILE // CHUNK

_sc_mesh = plsc.VectorSubcoreMesh(core_axis_name="c", subcore_axis_name="s")


DEG_W = 16


@functools.partial(
    pl.kernel,
    out_type=jax.ShapeDtypeStruct((4, NC, NPAD, DEG_W), jnp.float32),
    mesh=_sc_mesh,
    scratch_types=[
        pltpu.VMEM((NCHUNK, CHUNK), jnp.int32),
        pltpu.VMEM((CHUNK, DEG_W), jnp.float32),
        pltpu.VMEM((CHUNK, DEG_W), jnp.float32),
        pltpu.VMEM_SHARED((NPAD, DEG_W), jnp.float32),
    ],
)
def _degree_kernel(s1_hbm, d1_hbm, s2_hbm, d2_hbm, out_hbm,
                   idx_v, ones_v, zero_v, acc_sh):
    c = lax.axis_index("c")
    s = lax.axis_index("s")
    w = c * NS + s
    ones = jnp.ones((16,), jnp.float32)
    zeros = jnp.zeros((16,), jnp.float32)

    def fbody(i, _):
        ones_v[i, pl.ds(0, DEG_W)] = ones
        zero_v[i, pl.ds(0, DEG_W)] = zeros
        return 0

    lax.fori_loop(0, CHUNK, fbody, 0)

    row0 = s * ROWS_PER_TILE
    for t, e_hbm in enumerate((s1_hbm, d1_hbm, s2_hbm, d2_hbm)):
        pltpu.sync_copy(e_hbm.at[w], idx_v)
        for z in range(RCH):
            pltpu.sync_copy(zero_v, acc_sh.at[pl.ds(row0 + z * CHUNK, CHUNK)])
        plsc.subcore_barrier()

        def sbody(j, _):
            pltpu.sync_copy(ones_v, acc_sh.at[idx_v.at[j]], add=True)
            return 0

        lax.fori_loop(0, NCHUNK, sbody, 0)
        plsc.subcore_barrier()

        for z in range(RCH):
            pltpu.sync_copy(acc_sh.at[pl.ds(row0 + z * CHUNK, CHUNK)],
                            out_hbm.at[t, c, pl.ds(row0 + z * CHUNK, CHUNK)])


@functools.partial(
    pl.kernel,
    out_type=jax.ShapeDtypeStruct((NC, NPAD, DIM), jnp.float32),
    mesh=_sc_mesh,
    scratch_types=[
        pltpu.VMEM((NCHUNK, CHUNK), jnp.int32),
        pltpu.VMEM((NCHUNK, CHUNK), jnp.int32),
        pltpu.VMEM((CHUNK, DIM), jnp.float32),
        pltpu.VMEM_SHARED((NPAD, DIM), jnp.float32),
        pltpu.SemaphoreType.DMA,
    ],
)
def _mp_kernel(h_hbm, srcp_hbm, dstp_hbm, out_hbm, src_v, dst_v, buf_v, acc_sh, sem):
    c = lax.axis_index("c")
    s = lax.axis_index("s")
    w = c * NS + s
    pltpu.sync_copy(srcp_hbm.at[w], src_v)
    pltpu.sync_copy(dstp_hbm.at[w], dst_v)

    zeros = jnp.zeros((16,), jnp.float32)

    def zbody(i, _):
        buf_v[i // 8, pl.ds((i % 8) * 16, 16)] = zeros
        return 0

    lax.fori_loop(0, CHUNK * (DIM // 16), zbody, 0)

    row0 = s * ROWS_PER_TILE
    for z in range(RCH):
        pltpu.sync_copy(buf_v, acc_sh.at[pl.ds(row0 + z * CHUNK, CHUNK)])
    plsc.subcore_barrier()

    def mbody(j, _):
        pltpu.async_copy(h_hbm.at[src_v.at[j]], buf_v, sem).wait()
        pltpu.sync_copy(buf_v, acc_sh.at[dst_v.at[j]], add=True)
        return 0

    lax.fori_loop(0, NCHUNK, mbody, 0)
    plsc.subcore_barrier()

    for z in range(RCH):
        pltpu.sync_copy(acc_sh.at[pl.ds(row0 + z * CHUNK, CHUNK)],
                        out_hbm.at[c, pl.ds(row0 + z * CHUNK, CHUNK)])


def _tc0_body(feat_ref, degs_ref, degd_ref, w0_ref, h_out, ns_out, nd_out):
    degs = (degs_ref[0] + degs_ref[1])[:N, 0:1]
    degd = (degd_ref[0] + degd_ref[1])[:N, 0:1]
    ns = lax.rsqrt(jnp.maximum(degs, 1.0))
    nd = lax.rsqrt(jnp.maximum(degd, 1.0))
    f = feat_ref[...]
    iot = lax.broadcasted_iota(jnp.int32, (N, FEATURE_LEN), 1)
    h = jnp.zeros((N, FEATURE_LEN), jnp.float32)
    for j in range(L):
        h = h + (f[:, j][:, None] == iot).astype(jnp.float32)
    hs = jnp.dot(h, w0_ref[...], preferred_element_type=jnp.float32)
    h_out[...] = hs * ns
    ns_out[...] = ns
    nd_out[...] = nd


_tc0 = pl.pallas_call(
    _tc0_body,
    out_shape=[
        jax.ShapeDtypeStruct((N, DIM), jnp.float32),
        jax.ShapeDtypeStruct((N, 1), jnp.float32),
        jax.ShapeDtypeStruct((N, 1), jnp.float32),
    ],
)


def _tc1_body(acc_ref, nd_ref, ns_ref, b_ref, w_ref, out_ref):
    agg = (acc_ref[0] + acc_ref[1])[:N]
    h = jnp.maximum(agg * nd_ref[...] + b_ref[...][None, :], 0.0)
    out_ref[...] = jnp.dot(h, w_ref[...],
                           preferred_element_type=jnp.float32) * ns_ref[...]


_tc1 = pl.pallas_call(
    _tc1_body,
    out_shape=jax.ShapeDtypeStruct((N, DIM), jnp.float32),
)


def _tc2_body(acc_ref, nd_ref, b_ref, rs_out, nn_out):
    h = (acc_ref[0] + acc_ref[1])[:N] * nd_ref[...] + b_ref[...][None, :]
    rs_out[...] = jnp.sum(h, axis=0, keepdims=True)
    nn = jnp.sqrt(jnp.sum(h * h, axis=1))
    nn_out[...] = jnp.sum(nn)[None, None]


_tc2 = pl.pallas_call(
    _tc2_body,
    out_shape=[
        jax.ShapeDtypeStruct((1, DIM), jnp.float32),
        jax.ShapeDtypeStruct((1, 1), jnp.float32),
    ],
)


def _tc3_body(rs1_ref, nn1_ref, rs2_ref, wh_ref, bh_ref, wo_ref, bo_ref, out_ref):
    factor = jnp.sqrt(jnp.float32(DIM)) * N / nn1_ref[0, 0]
    emb = jnp.concatenate([rs1_ref[...], rs2_ref[...]], axis=1) * factor
    hid = jnp.maximum(
        jnp.dot(emb, wh_ref[...], preferred_element_type=jnp.float32)
        + bh_ref[...][None, :], 0.0)
    out_ref[...] = (jnp.dot(hid, wo_ref[...], preferred_element_type=jnp.float32)
                    + bo_ref[...][None, :])


_tc3 = pl.pallas_call(
    _tc3_body,
    out_shape=jax.ShapeDtypeStruct((1, 1), jnp.float32),
)


def _prep_edges(edge_index):
    src = edge_index[0].astype(jnp.int32).reshape(NW, EPW)
    dst = edge_index[1].astype(jnp.int32).reshape(NW, EPW)
    pad = EPW_PAD - EPW
    srcp = jnp.pad(src, ((0, 0), (0, pad)),
                   constant_values=0).reshape(NW, NCHUNK, CHUNK)
    srcd = jnp.pad(src, ((0, 0), (0, pad)),
                   constant_values=DUMMY).reshape(NW, NCHUNK, CHUNK)
    dstp = jnp.pad(dst, ((0, 0), (0, pad)),
                   constant_values=DUMMY).reshape(NW, NCHUNK, CHUNK)
    return srcp, srcd, dstp


def kernel(feature1, edge_index1, feature2, edge_index2,
           W0, b0, W1, b1, Wh, bh, Wo, bo):
    srcp1, srcd1, dstp1 = _prep_edges(edge_index1)
    srcp2, srcd2, dstp2 = _prep_edges(edge_index2)

    degp = _degree_kernel(srcd1, dstp1, srcd2, dstp2)

    hs0_1, ns1, nd1 = _tc0(feature1.astype(jnp.int32), degp[0], degp[1], W0)
    hs0_2, ns2, nd2 = _tc0(feature2.astype(jnp.int32), degp[2], degp[3], W0)

    acc1 = _mp_kernel(hs0_1, srcp1, dstp1)
    hs1_1 = _tc1(acc1, nd1, ns1, b0, W1)
    acc1b = _mp_kernel(hs1_1, srcp1, dstp1)
    rs1, nn1 = _tc2(acc1b, nd1, b1)

    acc2 = _mp_kernel(hs0_2, srcp2, dstp2)
    hs1_2 = _tc1(acc2, nd2, ns2, b0, W1)
    acc2b = _mp_kernel(hs1_2, srcp2, dstp2)
    rs2, nn2 = _tc2(acc2b, nd2, b1)

    return _tc3(rs1, nn1, rs2, Wh, bh, Wo, bo)
